# Initial kernel scaffold; baseline (speedup 1.0000x reference)
import jax
import jax.numpy as jnp
from jax.experimental import pallas as pl


def kernel(x, table):
    raise NotImplementedError("write your pallas kernel here")


# speedup vs baseline: 1.4661x; 1.4661x over previous
import functools
import math

import numpy as np
import jax
import jax.numpy as jnp
from jax import lax
from jax.experimental import pallas as pl
from jax.experimental.pallas import tpu as pltpu
from jax.experimental.pallas import tpu_sc as plsc

_VOCAB = 100000
_DIM = 768
_B, _L = 4, 4096
_SCALE = math.sqrt(_DIM)
_LANES = 16
_DV = _DIM // _LANES

_NC, _NS = 2, 16
_NW = _NC * _NS
_N = _B * _L
_PER_W = _N // _NW
_C = 64
_NCHUNK = _PER_W // _C


def _pe_table() -> np.ndarray:
    position = np.arange(_L, dtype=np.float32)[:, None]
    div_term = np.exp(
        np.arange(0, _DIM, 2, dtype=np.float32) * (-math.log(10000.0) / _DIM)
    )
    pe = np.zeros((_L, _DIM), dtype=np.float32)
    pe[:, 0::2] = np.sin(position * div_term)
    pe[:, 1::2] = np.cos(position * div_term)
    return pe


_PE = _pe_table()


@functools.partial(
    pl.kernel,
    mesh=plsc.VectorSubcoreMesh(core_axis_name="c", subcore_axis_name="s"),
    out_type=jax.ShapeDtypeStruct((_N, _DIM), jnp.float32),
    scratch_types=[
        pltpu.VMEM((_PER_W,), jnp.int32),
        pltpu.VMEM((_C, _DIM), jnp.float32),
        pltpu.VMEM((_C, _DIM), jnp.float32),
        pltpu.SemaphoreType.DMA,
    ],
)
def _embed(table_hbm, idx_hbm, pe_hbm, out_hbm, idx_v, rows_v, pe_v, sem):
    wid = lax.axis_index("s") * _NC + lax.axis_index("c")
    base = wid * _PER_W
    pltpu.sync_copy(idx_hbm.at[pl.ds(base, _PER_W)], idx_v)

    def chunk_body(ci, carry):
        g0 = base + ci * _C
        l0 = lax.rem(g0, _L)
        gather = pltpu.async_copy(
            table_hbm.at[idx_v.at[pl.ds(ci * _C, _C)]], rows_v, sem
        )
        pltpu.sync_copy(pe_hbm.at[pl.ds(l0, _C)], pe_v)
        gather.wait()

        def row_body(r, c2):
            for c in range(_DV):
                sl = (r, pl.ds(c * _LANES, _LANES))
                rows_v[sl] = rows_v[sl] * _SCALE + pe_v[sl]
            return c2

        lax.fori_loop(0, _C, row_body, 0)
        pltpu.sync_copy(rows_v, out_hbm.at[pl.ds(g0, _C)])
        return carry

    lax.fori_loop(0, _NCHUNK, chunk_body, 0)


def kernel(x, table):
    xf = x.reshape(-1).astype(jnp.int32)
    pe = jnp.asarray(_PE)
    out = _embed(table, xf, pe)
    return out.reshape(_B, _L, _DIM)


# speedup vs baseline: 1.6512x; 1.1262x over previous
import functools
import math

import numpy as np
import jax
import jax.numpy as jnp
from jax import lax
from jax.experimental import pallas as pl
from jax.experimental.pallas import tpu as pltpu
from jax.experimental.pallas import tpu_sc as plsc

_VOCAB = 100000
_DIM = 768
_B, _L = 4, 4096
_SCALE = math.sqrt(_DIM)
_LANES = 16
_DV = _DIM // _LANES

_NC, _NS = 2, 16
_NW = _NC * _NS
_N = _B * _L
_PPW = _L // _NW
_C = 32
_NP = _PPW // _C
_NPASS = 4
_PP = _PPW // _NPASS
_SH_ROWS = _NS * _PP


def _pe_table() -> np.ndarray:
    position = np.arange(_L, dtype=np.float32)[:, None]
    div_term = np.exp(
        np.arange(0, _DIM, 2, dtype=np.float32) * (-math.log(10000.0) / _DIM)
    )
    pe = np.zeros((_L, _DIM), dtype=np.float32)
    pe[:, 0::2] = np.sin(position * div_term)
    pe[:, 1::2] = np.cos(position * div_term)
    return pe


_PE = _pe_table()


@functools.partial(
    pl.kernel,
    mesh=plsc.VectorSubcoreMesh(core_axis_name="c", subcore_axis_name="s"),
    out_type=jax.ShapeDtypeStruct((_N, _DIM), jnp.float32),
    scratch_types=[
        pltpu.VMEM((_B * _PPW,), jnp.int32),
        pltpu.VMEM((_C, _DIM), jnp.float32),
        pltpu.VMEM((_C, _DIM), jnp.float32),
        pltpu.VMEM((_C, _DIM), jnp.float32),
        pltpu.VMEM((_C, _DIM), jnp.float32),
        pltpu.VMEM_SHARED((_SH_ROWS, _DIM), jnp.float32),
        pltpu.SemaphoreType.DMA,
        pltpu.SemaphoreType.DMA,
        pltpu.SemaphoreType.DMA,
        pltpu.SemaphoreType.DMA,
        pltpu.SemaphoreType.DMA,
        pltpu.SemaphoreType.DMA,
    ],
)
def _embed(table_hbm, idx_hbm, pe_hbm, out_hbm,
           idx_v, rows0, rows1, ob0, ob1, pe_sh,
           g0, g1, p0, p1, o0, o1):
    cid = lax.axis_index("c")
    sid = lax.axis_index("s")
    wid = cid * _NS + sid
    pbase = wid * _PPW

    rows = (rows0, rows1)
    ob = (ob0, ob1)
    gsem = (g0, g1)
    psem = (p0, p1)
    osem = (o0, o1)

    for b in range(_B):
        pltpu.sync_copy(
            idx_hbm.at[pl.ds(b * _L + pbase, _PPW)],
            idx_v.at[pl.ds(b * _PPW, _PPW)],
        )

    def issue(p_, b_, q_, ph_):
        pltpu.async_copy(
            table_hbm.at[idx_v.at[pl.ds(b_ * _PPW + p_ * _C, _C)]],
            rows[q_], gsem[q_],
        )
        loc = sid * _PP + (p_ - ph_ * (_NP // _NPASS)) * _C
        pltpu.async_copy(pe_sh.at[pl.ds(loc, _C)], ob[q_], psem[q_])

    def wait_gather(q_):
        pltpu.make_async_copy(table_hbm.at[pl.ds(0, _C)], rows[q_], gsem[q_]).wait()

    def wait_prefill(q_):
        pltpu.make_async_copy(pe_hbm.at[pl.ds(0, _C)], ob[q_], psem[q_]).wait()

    def wait_out(q_):
        pltpu.make_async_copy(ob[q_], out_hbm.at[pl.ds(0, _C)], osem[q_]).wait()

    def compute(q_):
        def row_body(r, acc):
            for cv in range(_DV):
                sl = pl.ds(cv * _LANES, _LANES)
                plsc.addupdate(ob[q_].at[r, sl], rows[q_][r, sl] * _SCALE)
            return acc
        lax.fori_loop(0, _C, row_body, 0)

    def issue_out(p_, b_, q_):
        row0 = b_ * _L + pbase + p_ * _C
        pltpu.async_copy(ob[q_], out_hbm.at[pl.ds(row0, _C)], osem[q_])

    for ph in range(_NPASS):
        plsc.subcore_barrier()
        pltpu.sync_copy(
            pe_hbm.at[pl.ds(pbase + ph * _PP, _PP)],
            pe_sh.at[pl.ds(sid * _PP, _PP)],
        )
        plsc.subcore_barrier()

        chunks = [(p, b)
                  for p in range(ph * (_NP // _NPASS), (ph + 1) * (_NP // _NPASS))
                  for b in range(_B)]
        issue(*chunks[0], 0, ph)
        for i, (p, b) in enumerate(chunks):
            q = i % 2
            if i >= 1:
                wait_out(q ^ 1)
            if i + 1 < len(chunks):
                issue(*chunks[i + 1], q ^ 1, ph)
            wait_gather(q)
            wait_prefill(q)
            compute(q)
            issue_out(p, b, q)
        wait_out(len(chunks) % 2 ^ 1)


def kernel(x, table):
    xf = x.reshape(-1).astype(jnp.int32)
    pe = jnp.asarray(_PE)
    out = _embed(table, xf, pe)
    return out.reshape(_B, _L, _DIM)


# speedup vs baseline: 2.2942x; 1.3894x over previous
import functools
import math

import numpy as np
import jax
import jax.numpy as jnp
from jax import lax
from jax.experimental import pallas as pl
from jax.experimental.pallas import tpu as pltpu
from jax.experimental.pallas import tpu_sc as plsc

_VOCAB = 100000
_DIM = 768
_B, _L = 4, 4096
_SCALE = math.sqrt(_DIM)
_LANES = 16
_DV = _DIM // _LANES

_NC, _NS = 2, 16
_NW = _NC * _NS
_N = _B * _L
_PPW = _L // _NW
_CP = 8
_CR = _B * _CP
_T = _PPW // _CP
_R = 4
_S = _T // _R


def _pe_table() -> np.ndarray:
    position = np.arange(_L, dtype=np.float32)[:, None]
    div_term = np.exp(
        np.arange(0, _DIM, 2, dtype=np.float32) * (-math.log(10000.0) / _DIM)
    )
    pe = np.zeros((_L, _DIM), dtype=np.float32)
    pe[:, 0::2] = np.sin(position * div_term)
    pe[:, 1::2] = np.cos(position * div_term)
    return pe


_PE = _pe_table()


@functools.partial(
    pl.kernel,
    mesh=plsc.VectorSubcoreMesh(core_axis_name="c", subcore_axis_name="s"),
    out_type=jax.ShapeDtypeStruct((_N, _DIM), jnp.float32),
    scratch_types=(
        [pltpu.VMEM((_B * _PPW,), jnp.int32)]
        + [pltpu.VMEM((_CR, _DIM), jnp.float32) for _ in range(_R)]
        + [pltpu.VMEM((_CP, _DIM), jnp.float32) for _ in range(_R)]
        + [pltpu.SemaphoreType.DMA for _ in range(3 * _R)]
    ),
)
def _embed(table_hbm, idx_hbm, pe_hbm, out_hbm, idx_v, *bufs):
    rows = bufs[0:_R]
    pe_v = bufs[_R:2 * _R]
    gsem = bufs[2 * _R:3 * _R]
    psem = bufs[3 * _R:4 * _R]
    osem = bufs[4 * _R:5 * _R]

    cid = lax.axis_index("c")
    sid = lax.axis_index("s")
    wid = cid * _NS + sid
    pbase = wid * _PPW

    for b in range(_B):
        pltpu.sync_copy(
            idx_hbm.at[pl.ds(b * _L + pbase, _PPW)],
            idx_v.at[pl.ds(b * _PPW, _PPW)],
        )

    def issue(t_, q_):
        for b in range(_B):
            pltpu.async_copy(
                table_hbm.at[idx_v.at[pl.ds(b * _PPW + t_ * _CP, _CP)]],
                rows[q_].at[pl.ds(b * _CP, _CP)],
                gsem[q_],
            )
        pltpu.async_copy(
            pe_hbm.at[pl.ds(pbase + t_ * _CP, _CP)], pe_v[q_], psem[q_]
        )

    def wait_in(q_):
        pltpu.make_async_copy(table_hbm.at[pl.ds(0, _CR)], rows[q_], gsem[q_]).wait()
        pltpu.make_async_copy(pe_hbm.at[pl.ds(0, _CP)], pe_v[q_], psem[q_]).wait()

    def wait_out(q_):
        pltpu.make_async_copy(rows[q_], out_hbm.at[pl.ds(0, _CR)], osem[q_]).wait()

    def compute(q_):
        def j_body(j, acc):
            for cv in range(_DV):
                sl = pl.ds(cv * _LANES, _LANES)
                pv = pe_v[q_][j, sl]
                for b in range(_B):
                    r = b * _CP + j
                    rows[q_][r, sl] = rows[q_][r, sl] * _SCALE + pv
            return acc
        lax.fori_loop(0, _CP, j_body, 0)

    def issue_out(t_, q_):
        for b in range(_B):
            pltpu.async_copy(
                rows[q_].at[pl.ds(b * _CP, _CP)],
                out_hbm.at[pl.ds(b * _L + pbase + t_ * _CP, _CP)],
                osem[q_],
            )

    issue(0, 0)
    issue(1, 1)

    def s_body(s, acc):
        for k in range(_R):
            t = s * _R + k
            q = k
            wait_in(q)
            compute(q)
            issue_out(t, q)
            qn = (k + 2) % _R
            if k < 2:
                @pl.when(s > 0)
                def _w(qn_=qn):
                    wait_out(qn_)
                issue(t + 2, qn)
            else:
                wait_out(qn)
                @pl.when(s < _S - 1)
                def _i(t_=t + 2, qn_=qn):
                    issue(t_, qn_)
        return acc

    lax.fori_loop(0, _S, s_body, 0)
    wait_out((_T - 2) % _R)
    wait_out((_T - 1) % _R)


def kernel(x, table):
    xf = x.reshape(-1).astype(jnp.int32)
    pe = jnp.asarray(_PE)
    out = _embed(table, xf, pe)
    return out.reshape(_B, _L, _DIM)
